# baseline (speedup 1.0000x reference)
import functools
import jax
import jax.numpy as jnp
from jax import lax
from jax.experimental import pallas as pl
from jax.experimental.pallas import tpu as pltpu
from jax.experimental.pallas import tpu_sc as plsc


def _bcast_body(w_ref, o_ref):
    o_ref[...] = jnp.broadcast_to(w_ref[...][None], o_ref.shape)


def _tc_part(pos_weight, batch, seq_len, embed_dim):
    bm = 1024
    grid = (seq_len // bm,)
    return pl.pallas_call(
        _bcast_body,
        grid=grid,
        in_specs=[pl.BlockSpec((bm, embed_dim), lambda i: (i, 0))],
        out_specs=pl.BlockSpec((batch, bm, embed_dim), lambda i: (0, i, 0)),
        out_shape=jax.ShapeDtypeStruct((batch, seq_len, embed_dim), pos_weight.dtype),
    )(pos_weight)


def _sc_part(pos_weight, seq_len, embed_dim):
    info = plsc.get_sparse_core_info()
    nc, ns = info.num_cores, info.num_subcores
    nw = nc * ns
    rows_per_w = seq_len // nw
    chunk = 32
    n_chunks = rows_per_w // chunk

    mesh = plsc.VectorSubcoreMesh(core_axis_name="c", subcore_axis_name="s")

    @functools.partial(
        pl.kernel,
        mesh=mesh,
        out_type=jax.ShapeDtypeStruct((seq_len, embed_dim), pos_weight.dtype),
        scratch_types=[
            pltpu.VMEM((chunk, embed_dim), pos_weight.dtype),
            pltpu.VMEM((chunk, embed_dim), pos_weight.dtype),
            pltpu.SemaphoreType.DMA,
            pltpu.SemaphoreType.DMA,
        ],
    )
    def sc_copy(table_hbm, out_hbm, buf0, buf1, sem_in, sem_out):
        wid = lax.axis_index("s") * nc + lax.axis_index("c")
        base = wid * rows_per_w
        bufs = (buf0, buf1)

        reads = [None] * n_chunks
        writes = [None] * n_chunks
        reads[0] = pltpu.async_copy(
            table_hbm.at[pl.ds(base, chunk), :], bufs[0], sem_in
        )
        for c in range(n_chunks):
            buf = bufs[c % 2]
            reads[c].wait()
            writes[c] = pltpu.async_copy(
                buf, out_hbm.at[pl.ds(base + c * chunk, chunk), :], sem_out
            )
            if c + 1 < n_chunks:
                if c >= 1:
                    writes[c - 1].wait()
                reads[c + 1] = pltpu.async_copy(
                    table_hbm.at[pl.ds(base + (c + 1) * chunk, chunk), :],
                    bufs[(c + 1) % 2],
                    sem_in,
                )
        writes[n_chunks - 2].wait()
        writes[n_chunks - 1].wait()

    return sc_copy(pos_weight)


def kernel(x, pos_weight):
    batch, seq_len = x.shape
    embed_dim = pos_weight.shape[1]

    tc = _tc_part(pos_weight, batch - 1, seq_len, embed_dim)
    sc = _sc_part(pos_weight, seq_len, embed_dim)
    return jnp.concatenate([tc, sc[None]], axis=0)


# speedup vs baseline: 3.2939x; 3.2939x over previous
import jax
import jax.numpy as jnp
from jax.experimental import pallas as pl
from jax.experimental.pallas import tpu as pltpu

_CHUNK = 2048
_K = 3


def _dma_body(w_hbm, o_hbm, b0, b1, b2, rsem, wsem):
    batch, seq_len, _ = o_hbm.shape
    bufs = (b0, b1, b2)
    n_chunks = seq_len // _CHUNK

    def read(c):
        return pltpu.async_copy(
            w_hbm.at[pl.ds(c * _CHUNK, _CHUNK), :], bufs[c % _K], rsem.at[c % _K]
        )

    reads = {}
    writes = {}
    for c in range(min(_K, n_chunks)):
        reads[c] = read(c)
    for c in range(n_chunks):
        k = c % _K
        reads[c].wait()
        writes[c] = [
            pltpu.async_copy(
                bufs[k], o_hbm.at[b, pl.ds(c * _CHUNK, _CHUNK), :], wsem.at[k]
            )
            for b in range(batch)
        ]
        if c + _K < n_chunks:
            for h in writes[c]:
                h.wait()
            reads[c + _K] = read(c + _K)
    for c in range(max(0, n_chunks - _K), n_chunks):
        if c in writes:
            for h in writes[c]:
                h.wait()


def kernel(x, pos_weight):
    batch, seq_len = x.shape
    embed_dim = pos_weight.shape[1]

    out = pl.pallas_call(
        _dma_body,
        in_specs=[pl.BlockSpec(memory_space=pl.ANY)],
        out_specs=pl.BlockSpec(memory_space=pl.ANY),
        out_shape=jax.ShapeDtypeStruct((batch, seq_len, embed_dim), pos_weight.dtype),
        scratch_shapes=[pltpu.VMEM((_CHUNK, embed_dim), pos_weight.dtype)] * _K
        + [pltpu.SemaphoreType.DMA((_K,)), pltpu.SemaphoreType.DMA((_K,))],
    )(pos_weight)
    return out


# speedup vs baseline: 3.3277x; 1.0103x over previous
import jax
import jax.numpy as jnp
from jax.experimental import pallas as pl
from jax.experimental.pallas import tpu as pltpu

_CHUNK = 4096
_K = 2


def _dma_body(w_hbm, o_hbm, b0, b1, rsem, wsem):
    batch, seq_len, _ = o_hbm.shape
    bufs = (b0, b1)
    n_chunks = seq_len // _CHUNK

    def read(c):
        return pltpu.async_copy(
            w_hbm.at[pl.ds(c * _CHUNK, _CHUNK), :], bufs[c % _K], rsem.at[c % _K]
        )

    reads = {}
    writes = {}
    for c in range(min(_K, n_chunks)):
        reads[c] = read(c)
    for c in range(n_chunks):
        k = c % _K
        reads[c].wait()
        writes[c] = [
            pltpu.async_copy(
                bufs[k], o_hbm.at[b, pl.ds(c * _CHUNK, _CHUNK), :], wsem.at[k]
            )
            for b in range(batch)
        ]
        if c + _K < n_chunks:
            for h in writes[c]:
                h.wait()
            reads[c + _K] = read(c + _K)
    for c in range(max(0, n_chunks - _K), n_chunks):
        if c in writes:
            for h in writes[c]:
                h.wait()


def kernel(x, pos_weight):
    batch, seq_len = x.shape
    embed_dim = pos_weight.shape[1]

    out = pl.pallas_call(
        _dma_body,
        in_specs=[pl.BlockSpec(memory_space=pl.ANY)],
        out_specs=pl.BlockSpec(memory_space=pl.ANY),
        out_shape=jax.ShapeDtypeStruct((batch, seq_len, embed_dim), pos_weight.dtype),
        scratch_shapes=[pltpu.VMEM((_CHUNK, embed_dim), pos_weight.dtype)] * _K
        + [pltpu.SemaphoreType.DMA((_K,)), pltpu.SemaphoreType.DMA((_K,))],
    )(pos_weight)
    return out
